# baseline (speedup 1.0000x reference)
import functools

import jax
import jax.numpy as jnp
from jax import lax
from jax.experimental import pallas as pl
from jax.experimental.pallas import tpu as pltpu
from jax.experimental.pallas import tpu_sc as plsc

NC = 2
NS = 16
NW = NC * NS

CH = 32
NBUF = 2


def _sc_broadcast(B, L, D):
    rows_per_w = L // NW
    n_chunks = rows_per_w // CH
    mesh = plsc.VectorSubcoreMesh(core_axis_name="c", subcore_axis_name="s")

    @functools.partial(
        pl.kernel,
        mesh=mesh,
        out_type=jax.ShapeDtypeStruct((B, L, D), jnp.float32),
        scratch_types=[
            pltpu.VMEM((NBUF, CH, D), jnp.float32),
            pltpu.VMEM_SHARED((NS, NBUF, CH, D), jnp.float32),
            pltpu.SemaphoreType.DMA,
            pltpu.SemaphoreType.DMA,
            pltpu.SemaphoreType.DMA,
            pltpu.SemaphoreType.DMA,
        ],
    )
    def k(emb_hbm, out_hbm, buf, shbuf, sem_in, sem_out, sem_sin, sem_sout):
        sid = lax.axis_index("s")
        wid = lax.axis_index("c") * NS + sid
        base = wid * rows_per_w

        fetch = [
            pltpu.make_async_copy(
                emb_hbm.at[pl.ds(base + i * CH, CH), :], buf.at[i % NBUF], sem_in
            )
            for i in range(n_chunks)
        ]
        sfetch = [
            pltpu.make_async_copy(
                emb_hbm.at[pl.ds(base + i * CH, CH), :],
                shbuf.at[sid, i % NBUF],
                sem_sin,
            )
            for i in range(n_chunks)
        ]
        stores = [
            [
                pltpu.make_async_copy(
                    buf.at[i % NBUF],
                    out_hbm.at[b, pl.ds(base + i * CH, CH), :],
                    sem_out,
                )
                for b in range(B // 2)
            ]
            for i in range(n_chunks)
        ]
        sstores = [
            [
                pltpu.make_async_copy(
                    shbuf.at[sid, i % NBUF],
                    out_hbm.at[b, pl.ds(base + i * CH, CH), :],
                    sem_sout,
                )
                for b in range(B // 2, B)
            ]
            for i in range(n_chunks)
        ]

        for i in range(NBUF - 1):
            fetch[i].start()
            sfetch[i].start()
        for i in range(n_chunks):
            fetch[i].wait()
            sfetch[i].wait()
            if i + NBUF - 1 < n_chunks:
                if i >= 1:
                    for c in stores[i - 1]:
                        c.wait()
                    for c in sstores[i - 1]:
                        c.wait()
                fetch[i + NBUF - 1].start()
                sfetch[i + NBUF - 1].start()
            for c in stores[i]:
                c.start()
            for c in sstores[i]:
                c.start()
        for i in range(max(0, n_chunks - NBUF), n_chunks):
            for c in stores[i]:
                c.wait()
            for c in sstores[i]:
                c.wait()

    return k


def kernel(x, emb):
    B, L = x.shape[0], x.shape[1]
    D = emb.shape[1]
    return _sc_broadcast(B, L, D)(emb)


# speedup vs baseline: 1.2559x; 1.2559x over previous
import functools

import jax
import jax.numpy as jnp
from jax import lax
from jax.experimental import pallas as pl
from jax.experimental.pallas import tpu as pltpu
from jax.experimental.pallas import tpu_sc as plsc

NC = 2
NS = 16
NW = NC * NS

CH = 64
NBUF = 2


def _sc_broadcast(B, L, D):
    rows_per_w = L // NW
    n_chunks = rows_per_w // CH
    mesh = plsc.VectorSubcoreMesh(core_axis_name="c", subcore_axis_name="s")

    @functools.partial(
        pl.kernel,
        mesh=mesh,
        out_type=jax.ShapeDtypeStruct((B, L, D), jnp.float32),
        scratch_types=[
            pltpu.VMEM((NBUF, CH, D), jnp.float32),
            pltpu.SemaphoreType.DMA,
            pltpu.SemaphoreType.DMA,
        ],
    )
    def k(emb_hbm, out_hbm, buf, sem_in, sem_out):
        wid = lax.axis_index("c") * NS + lax.axis_index("s")
        base = wid * rows_per_w

        fetch = [
            pltpu.make_async_copy(
                emb_hbm.at[pl.ds(base + i * CH, CH), :], buf.at[i % NBUF], sem_in
            )
            for i in range(n_chunks)
        ]
        stores = [
            [
                pltpu.make_async_copy(
                    buf.at[i % NBUF],
                    out_hbm.at[b, pl.ds(base + i * CH, CH), :],
                    sem_out,
                )
                for b in range(B)
            ]
            for i in range(n_chunks)
        ]

        for i in range(NBUF - 1):
            fetch[i].start()
        for i in range(n_chunks):
            fetch[i].wait()
            if i + NBUF - 1 < n_chunks:
                if i >= 1:
                    for c in stores[i - 1]:
                        c.wait()
                fetch[i + NBUF - 1].start()
            for c in stores[i]:
                c.start()
        for i in range(max(0, n_chunks - NBUF), n_chunks):
            for c in stores[i]:
                c.wait()

    return k


def kernel(x, emb):
    B, L = x.shape[0], x.shape[1]
    D = emb.shape[1]
    return _sc_broadcast(B, L, D)(emb)
